# baseline (speedup 1.0000x reference)
import jax
import jax.numpy as jnp
from jax.experimental import pallas as pl
from jax.experimental.pallas import tpu as pltpu

_BB = 32


def _chunks(arr2d, r, n):
    return jnp.concatenate(
        [
            jax.lax.slice(arr2d, (r, t * 128), (r + 1, (t + 1) * 128))
            for t in range(n)
        ],
        axis=0,
    )


def _pe_kernel(row_ref, col_ref, o_ref, scratch_ref, sem):
    rows = row_ref[...]
    cols = col_ref[...]
    n = cols.shape[1] // 128
    col0 = _chunks(cols, 0, n)
    col1 = _chunks(cols, 1, n)
    row0 = _chunks(rows, 0, n)
    row1 = _chunks(rows, 1, n)
    m00 = jnp.concatenate([col0, row0], axis=0)
    m01 = jnp.concatenate([col1, row0], axis=0)
    m10 = jnp.concatenate([col0, row1], axis=0)
    m11 = jnp.concatenate([col1, row1], axis=0)
    c0 = jnp.stack([m00, m01], axis=1).reshape(2 * m00.shape[0], 128)
    c1 = jnp.stack([m10, m11], axis=1).reshape(2 * m00.shape[0], 128)
    cc = jnp.stack([c0, c1], axis=0)
    scratch_ref[...] = jnp.broadcast_to(cc[None], scratch_ref.shape)

    b = o_ref.shape[0]
    copies = [
        pltpu.make_async_copy(
            scratch_ref, o_ref.at[pl.ds(t * _BB, _BB)], sem
        )
        for t in range(b // _BB)
    ]
    for c in copies:
        c.start()
    for c in copies:
        c.wait()


def kernel(x, row_embed, col_embed):
    b, _, h, w = x.shape
    d = row_embed.shape[1]
    ntj = 2 * d * w // 128
    out = pl.pallas_call(
        _pe_kernel,
        in_specs=[
            pl.BlockSpec(memory_space=pltpu.MemorySpace.VMEM),
            pl.BlockSpec(memory_space=pltpu.MemorySpace.VMEM),
        ],
        out_specs=pl.BlockSpec(memory_space=pl.ANY),
        out_shape=jax.ShapeDtypeStruct((b, h, ntj, 128), x.dtype),
        scratch_shapes=[
            pltpu.VMEM((_BB, h, ntj, 128), jnp.float32),
            pltpu.SemaphoreType.DMA,
        ],
    )(row_embed, col_embed)
    out5 = out.reshape(b, h, ntj // w, w, 128)
    return out5.transpose(0, 2, 4, 1, 3).reshape(b, 2 * d, h, w)


# speedup vs baseline: 1.0389x; 1.0389x over previous
import jax
import jax.numpy as jnp
from jax.experimental import pallas as pl
from jax.experimental.pallas import tpu as pltpu

_BB = 8


def _chunks(arr2d, r, n):
    return jnp.concatenate(
        [
            jax.lax.slice(arr2d, (r, t * 128), (r + 1, (t + 1) * 128))
            for t in range(n)
        ],
        axis=0,
    )


def _pe_kernel(row_ref, col_ref, o_ref, scratch_ref, sem):
    rows = row_ref[...]
    cols = col_ref[...]
    n = cols.shape[1] // 128
    col0 = _chunks(cols, 0, n)
    col1 = _chunks(cols, 1, n)
    row0 = _chunks(rows, 0, n)
    row1 = _chunks(rows, 1, n)
    m00 = jnp.concatenate([col0, row0], axis=0)
    m01 = jnp.concatenate([col1, row0], axis=0)
    m10 = jnp.concatenate([col0, row1], axis=0)
    m11 = jnp.concatenate([col1, row1], axis=0)
    c0 = jnp.stack([m00, m01], axis=1).reshape(2 * m00.shape[0], 128)
    c1 = jnp.stack([m10, m11], axis=1).reshape(2 * m00.shape[0], 128)
    cc = jnp.stack([c0, c1], axis=0)
    scratch_ref[...] = jnp.broadcast_to(cc[None], scratch_ref.shape)

    b = o_ref.shape[0]
    copies = [
        pltpu.make_async_copy(
            scratch_ref, o_ref.at[pl.ds(t * _BB, _BB)], sem
        )
        for t in range(b // _BB)
    ]
    for c in copies:
        c.start()
    for c in copies:
        c.wait()


def kernel(x, row_embed, col_embed):
    b, _, h, w = x.shape
    d = row_embed.shape[1]
    ntj = 2 * d * w // 128
    out = pl.pallas_call(
        _pe_kernel,
        in_specs=[
            pl.BlockSpec(memory_space=pltpu.MemorySpace.VMEM),
            pl.BlockSpec(memory_space=pltpu.MemorySpace.VMEM),
        ],
        out_specs=pl.BlockSpec(memory_space=pl.ANY),
        out_shape=jax.ShapeDtypeStruct((b, h, ntj, 128), x.dtype),
        scratch_shapes=[
            pltpu.VMEM((_BB, h, ntj, 128), jnp.float32),
            pltpu.SemaphoreType.DMA,
        ],
    )(row_embed, col_embed)
    out5 = out.reshape(b, h, ntj // w, w, 128)
    return out5.transpose(0, 2, 4, 1, 3).reshape(b, 2 * d, h, w)


# speedup vs baseline: 1.0439x; 1.0048x over previous
import jax
import jax.numpy as jnp
from jax.experimental import pallas as pl
from jax.experimental.pallas import tpu as pltpu

_BB = 4


def _chunks(arr2d, r, n):
    return jnp.concatenate(
        [
            jax.lax.slice(arr2d, (r, t * 128), (r + 1, (t + 1) * 128))
            for t in range(n)
        ],
        axis=0,
    )


def _pe_kernel(row_ref, col_ref, o_ref, scratch_ref, sem):
    rows = row_ref[...]
    cols = col_ref[...]
    n = cols.shape[1] // 128
    col0 = _chunks(cols, 0, n)
    col1 = _chunks(cols, 1, n)
    row0 = _chunks(rows, 0, n)
    row1 = _chunks(rows, 1, n)
    m00 = jnp.concatenate([col0, row0], axis=0)
    m01 = jnp.concatenate([col1, row0], axis=0)
    m10 = jnp.concatenate([col0, row1], axis=0)
    m11 = jnp.concatenate([col1, row1], axis=0)
    c0 = jnp.stack([m00, m01], axis=1).reshape(2 * m00.shape[0], 128)
    c1 = jnp.stack([m10, m11], axis=1).reshape(2 * m00.shape[0], 128)
    cc = jnp.stack([c0, c1], axis=0)
    scratch_ref[...] = jnp.broadcast_to(cc[None], scratch_ref.shape)

    b = o_ref.shape[0]
    copies = [
        pltpu.make_async_copy(
            scratch_ref, o_ref.at[pl.ds(t * _BB, _BB)], sem
        )
        for t in range(b // _BB)
    ]
    for c in copies:
        c.start()
    for c in copies:
        c.wait()


def kernel(x, row_embed, col_embed):
    b, _, h, w = x.shape
    d = row_embed.shape[1]
    ntj = 2 * d * w // 128
    out = pl.pallas_call(
        _pe_kernel,
        in_specs=[
            pl.BlockSpec(memory_space=pltpu.MemorySpace.VMEM),
            pl.BlockSpec(memory_space=pltpu.MemorySpace.VMEM),
        ],
        out_specs=pl.BlockSpec(memory_space=pl.ANY),
        out_shape=jax.ShapeDtypeStruct((b, h, ntj, 128), x.dtype),
        scratch_shapes=[
            pltpu.VMEM((_BB, h, ntj, 128), jnp.float32),
            pltpu.SemaphoreType.DMA,
        ],
    )(row_embed, col_embed)
    out5 = out.reshape(b, h, ntj // w, w, 128)
    return out5.transpose(0, 2, 4, 1, 3).reshape(b, 2 * d, h, w)


# speedup vs baseline: 1.0463x; 1.0023x over previous
import jax
import jax.numpy as jnp
from jax.experimental import pallas as pl
from jax.experimental.pallas import tpu as pltpu

_BB = 2


def _chunks(arr2d, r, n):
    return jnp.concatenate(
        [
            jax.lax.slice(arr2d, (r, t * 128), (r + 1, (t + 1) * 128))
            for t in range(n)
        ],
        axis=0,
    )


def _pe_kernel(row_ref, col_ref, o_ref, scratch_ref, sem):
    rows = row_ref[...]
    cols = col_ref[...]
    n = cols.shape[1] // 128
    col0 = _chunks(cols, 0, n)
    col1 = _chunks(cols, 1, n)
    row0 = _chunks(rows, 0, n)
    row1 = _chunks(rows, 1, n)
    m00 = jnp.concatenate([col0, row0], axis=0)
    m01 = jnp.concatenate([col1, row0], axis=0)
    m10 = jnp.concatenate([col0, row1], axis=0)
    m11 = jnp.concatenate([col1, row1], axis=0)
    c0 = jnp.stack([m00, m01], axis=1).reshape(2 * m00.shape[0], 128)
    c1 = jnp.stack([m10, m11], axis=1).reshape(2 * m00.shape[0], 128)
    cc = jnp.stack([c0, c1], axis=0)
    scratch_ref[...] = jnp.broadcast_to(cc[None], scratch_ref.shape)

    b = o_ref.shape[0]
    copies = [
        pltpu.make_async_copy(
            scratch_ref, o_ref.at[pl.ds(t * _BB, _BB)], sem
        )
        for t in range(b // _BB)
    ]
    for c in copies:
        c.start()
    for c in copies:
        c.wait()


def kernel(x, row_embed, col_embed):
    b, _, h, w = x.shape
    d = row_embed.shape[1]
    ntj = 2 * d * w // 128
    out = pl.pallas_call(
        _pe_kernel,
        in_specs=[
            pl.BlockSpec(memory_space=pltpu.MemorySpace.VMEM),
            pl.BlockSpec(memory_space=pltpu.MemorySpace.VMEM),
        ],
        out_specs=pl.BlockSpec(memory_space=pl.ANY),
        out_shape=jax.ShapeDtypeStruct((b, h, ntj, 128), x.dtype),
        scratch_shapes=[
            pltpu.VMEM((_BB, h, ntj, 128), jnp.float32),
            pltpu.SemaphoreType.DMA,
        ],
    )(row_embed, col_embed)
    out5 = out.reshape(b, h, ntj // w, w, 128)
    return out5.transpose(0, 2, 4, 1, 3).reshape(b, 2 * d, h, w)
